# baseline (device time: 433696 ns/iter reference)
import jax
import jax.numpy as jnp
from jax import lax
from jax.experimental import pallas as pl
from jax.experimental.pallas import tpu as pltpu

M, N = 16384, 1024
HALF = M // 2
MC = 512
NC = HALF // MC
S = 6
AHEAD = 3


def kernel(x):
    def body(x_hbm, out_hbm, my_buf, yrecv_buf, local_sems,
             out_sems, y_send_sems, y_recv_sems, x_send_sems, x_recv_sems):
        my_x = lax.axis_index("x")
        my_y = lax.axis_index("y")
        nbr_y = (my_x, 1 - my_y)
        nbr_x = (1 - my_x, my_y)
        base = my_x * HALF

        barrier_sem = pltpu.get_barrier_semaphore()
        for nbr in (nbr_y, nbr_x):
            pl.semaphore_signal(
                barrier_sem, inc=1, device_id=nbr,
                device_id_type=pl.DeviceIdType.MESH,
            )
        pl.semaphore_wait(barrier_sem, 2)

        def rows(c):
            return pl.ds(base + c * MC, MC)

        def make_y(c):
            s = c % S
            return pltpu.make_async_remote_copy(
                src_ref=x_hbm.at[rows(c), :],
                dst_ref=yrecv_buf.at[s],
                send_sem=y_send_sems.at[s],
                recv_sem=y_recv_sems.at[s],
                device_id=nbr_y,
                device_id_type=pl.DeviceIdType.MESH,
            )

        def make_x(c):
            s = c % S
            return pltpu.make_async_remote_copy(
                src_ref=my_buf.at[s],
                dst_ref=out_hbm.at[rows(c), :],
                send_sem=x_send_sems.at[s],
                recv_sem=x_recv_sems.at[s],
                device_id=nbr_x,
                device_id_type=pl.DeviceIdType.MESH,
            )

        def make_local(c):
            s = c % S
            return pltpu.make_async_copy(
                x_hbm.at[rows(c), :], my_buf.at[s], local_sems.at[s]
            )

        y_rdma, x_rdma, loc, cp_out = {}, {}, {}, {}

        for c in range(AHEAD):
            y_rdma[c] = make_y(c)
            y_rdma[c].start()
            loc[c] = make_local(c)
            loc[c].start()

        for c in range(NC):
            s = c % S
            loc[c].wait()
            y_rdma[c].wait_recv()
            y_rdma[c].wait_send()
            if c >= S - AHEAD:
                cp_out[c - (S - AHEAD)].wait()
            if c + AHEAD < NC:
                y_rdma[c + AHEAD] = make_y(c + AHEAD)
                y_rdma[c + AHEAD].start()
                loc[c + AHEAD] = make_local(c + AHEAD)
                loc[c + AHEAD].start()
            if c >= 2:
                x_rdma[c - 2].wait_recv()
                x_rdma[c - 2].wait_send()
            my_buf[s] = my_buf[s] + yrecv_buf[s]
            x_rdma[c] = make_x(c)
            x_rdma[c].start()
            cp_out[c] = pltpu.make_async_copy(
                my_buf.at[s], out_hbm.at[rows(c), :], out_sems.at[s]
            )
            cp_out[c].start()

        for c in (NC - 2, NC - 1):
            x_rdma[c].wait_recv()
            x_rdma[c].wait_send()
        for c in range(NC - (S - AHEAD), NC):
            cp_out[c].wait()

    out_shape = jax.ShapeDtypeStruct((M, N), jnp.float32)
    return pl.pallas_call(
        body,
        out_shape=out_shape,
        in_specs=[pl.BlockSpec(memory_space=pl.ANY)],
        out_specs=pl.BlockSpec(memory_space=pl.ANY),
        scratch_shapes=[
            pltpu.VMEM((S, MC, N), jnp.float32),
            pltpu.VMEM((S, MC, N), jnp.float32),
            pltpu.SemaphoreType.DMA((S,)),
            pltpu.SemaphoreType.DMA((S,)),
            pltpu.SemaphoreType.DMA((S,)),
            pltpu.SemaphoreType.DMA((S,)),
            pltpu.SemaphoreType.DMA((S,)),
            pltpu.SemaphoreType.DMA((S,)),
        ],
        compiler_params=pltpu.CompilerParams(collective_id=0),
    )(x)
